# Initial kernel scaffold; baseline (speedup 1.0000x reference)
import jax
import jax.numpy as jnp
from jax.experimental import pallas as pl


def kernel(x, edge_index, W1l, W1r, b1, W2l, W2r, b2):
    raise NotImplementedError("write your pallas kernel here")


# speedup vs baseline: 8.7200x; 8.7200x over previous
import functools

import jax
import jax.numpy as jnp
from jax import lax
from jax.experimental import pallas as pl
from jax.experimental.pallas import tpu as pltpu
from jax.experimental.pallas import tpu_sc as plsc

N = 10000
D = 128
E = 320000
NC = 2
NS = 16
NW = NC * NS
ROWS = E // 128
RPW = ROWS // NW
EXTRA = ROWS - RPW * NW
IJ = 13
NBLK = RPW // IJ
RPT = N // NS
ZC = 125


def _seg_sum_builder(with_cnt):
  mesh = plsc.VectorSubcoreMesh(core_axis_name="c", subcore_axis_name="s")
  out_type = [jax.ShapeDtypeStruct((NC, N, D), jnp.float32)]
  scratch = [
      pltpu.VMEM((IJ, 128), jnp.int32),
      pltpu.VMEM((IJ, 128), jnp.int32),
      pltpu.VMEM((128, D), jnp.float32),
      pltpu.VMEM_SHARED((N, D), jnp.float32),
      pltpu.SemaphoreType.DMA,
  ]
  if with_cnt:
    out_type.append(jax.ShapeDtypeStruct((NC, N), jnp.float32))
    scratch += [
        pltpu.VMEM((128,), jnp.float32),
        pltpu.VMEM((N,), jnp.float32),
        pltpu.VMEM_SHARED((N,), jnp.float32),
    ]

  def body(x_hbm, src_hbm, dst_hbm, agg_out, *rest):
    if with_cnt:
      (cnt_out, sidx, didx, rows, agg_sh, sem, ones_v, cnt_v, cnt_sh) = rest
    else:
      (sidx, didx, rows, agg_sh, sem) = rest
    c = lax.axis_index("c")
    s = lax.axis_index("s")
    w = c * NS + s
    zero16 = jnp.zeros((16,), jnp.float32)

    @pl.loop(0, ZC)
    def _(r):
      for j in range(D // 16):
        rows[r, pl.ds(j * 16, 16)] = zero16

    for j in range(RPT // ZC):
      pltpu.sync_copy(rows.at[pl.ds(0, ZC)],
                      agg_sh.at[pl.ds(s * RPT + j * ZC, ZC)])

    if with_cnt:
      one16 = jnp.ones((16,), jnp.float32)

      @pl.loop(0, 128 // 16)
      def _(i):
        ones_v[pl.ds(i * 16, 16)] = one16

      @pl.when(s == 0)
      def _():
        @pl.loop(0, N // 16)
        def _(i):
          cnt_v[pl.ds(i * 16, 16)] = zero16
        pltpu.sync_copy(cnt_v, cnt_sh)

    plsc.subcore_barrier()

    def do_block(idx_s, idx_d):
      pltpu.async_copy(x_hbm.at[idx_s], rows, sem).wait()
      pltpu.sync_copy(rows, agg_sh.at[idx_d], add=True)
      if with_cnt:
        pltpu.sync_copy(ones_v, cnt_sh.at[idx_d], add=True)

    base = w * RPW

    @pl.loop(0, NBLK)
    def _(g):
      off = base + g * IJ
      pltpu.sync_copy(src_hbm.at[pl.ds(off, IJ)], sidx)
      pltpu.sync_copy(dst_hbm.at[pl.ds(off, IJ)], didx)
      for j in range(IJ):
        do_block(sidx.at[j], didx.at[j])

    @pl.when(w < EXTRA)
    def _():
      off = RPW * NW + w
      pltpu.sync_copy(src_hbm.at[pl.ds(off, 1)], sidx.at[pl.ds(0, 1)])
      pltpu.sync_copy(dst_hbm.at[pl.ds(off, 1)], didx.at[pl.ds(0, 1)])
      do_block(sidx.at[0], didx.at[0])

    plsc.subcore_barrier()

    for j in range(RPT // ZC):
      r0 = s * RPT + j * ZC
      pltpu.sync_copy(agg_sh.at[pl.ds(r0, ZC)], rows.at[pl.ds(0, ZC)])
      pltpu.sync_copy(rows.at[pl.ds(0, ZC)], agg_out.at[c, pl.ds(r0, ZC)])
    if with_cnt:
      @pl.when(s == 0)
      def _():
        pltpu.sync_copy(cnt_sh, cnt_v)
        pltpu.sync_copy(cnt_v, cnt_out.at[c])

  return pl.kernel(
      body, out_type=out_type, mesh=mesh, scratch_types=scratch,
      compiler_params=pltpu.CompilerParams(use_tc_tiling_on_sc=False))


_seg_sum_cnt = _seg_sum_builder(True)
_seg_sum = _seg_sum_builder(False)

_BLK = 1000


def _layer_body(aggp, cntp, x, wlT, wrT, b, o, *, relu):
  agg = aggp[0] + aggp[1]
  cnt = jnp.maximum(cntp[0] + cntp[1], 1.0)
  mean = agg / cnt
  h = (jnp.dot(mean, wlT[...], preferred_element_type=jnp.float32)
       + jnp.dot(x[...], wrT[...], preferred_element_type=jnp.float32)
       + b[...])
  if relu:
    h = jnp.maximum(h, 0.0)
  o[...] = h


def _layer(aggp, cntp, xin, wlT, wrT, b, relu):
  return pl.pallas_call(
      functools.partial(_layer_body, relu=relu),
      grid=(N // _BLK,),
      in_specs=[
          pl.BlockSpec((NC, _BLK, D), lambda i: (0, i, 0)),
          pl.BlockSpec((NC, _BLK, 1), lambda i: (0, i, 0)),
          pl.BlockSpec((_BLK, D), lambda i: (i, 0)),
          pl.BlockSpec((D, D), lambda i: (0, 0)),
          pl.BlockSpec((D, D), lambda i: (0, 0)),
          pl.BlockSpec((1, D), lambda i: (0, 0)),
      ],
      out_specs=pl.BlockSpec((_BLK, D), lambda i: (i, 0)),
      out_shape=jax.ShapeDtypeStruct((N, D), jnp.float32),
  )(aggp, cntp, xin, wlT, wrT, b)


def kernel(x, edge_index, W1l, W1r, b1, W2l, W2r, b2):
  src = edge_index[0].reshape(ROWS, 128)
  dst = edge_index[1].reshape(ROWS, 128)
  agg1, cntp = _seg_sum_cnt(x, src, dst)
  cntp = cntp.reshape(NC, N, 1)
  h = _layer(agg1, cntp, x, W1l.T, W1r.T, b1.reshape(1, D), True)
  (agg2,) = _seg_sum(h, src, dst)
  return _layer(agg2, cntp, h, W2l.T, W2r.T, b2.reshape(1, D), False)


# speedup vs baseline: 11.0236x; 1.2642x over previous
import functools

import jax
import jax.numpy as jnp
from jax import lax
from jax.experimental import pallas as pl
from jax.experimental.pallas import tpu as pltpu
from jax.experimental.pallas import tpu_sc as plsc

N = 10000
D = 128
E = 320000
NC = 2
NS = 16
NW = NC * NS
ROWS = E // 128
RPW = ROWS // NW
EXTRA = ROWS - RPW * NW
IJ = 13
NBLK = RPW // IJ
RPT = N // NS
ZC = 125


def _seg_sum_builder(with_cnt):
  mesh = plsc.VectorSubcoreMesh(core_axis_name="c", subcore_axis_name="s")
  out_type = [jax.ShapeDtypeStruct((NC, N, D), jnp.float32)]
  scratch = [
      pltpu.VMEM((IJ, 128), jnp.int32),
      pltpu.VMEM((IJ, 128), jnp.int32),
      pltpu.VMEM((128, D), jnp.float32),
      pltpu.VMEM((128, D), jnp.float32),
      pltpu.VMEM_SHARED((N, D), jnp.float32),
      pltpu.SemaphoreType.DMA,
      pltpu.SemaphoreType.DMA,
  ]
  if with_cnt:
    out_type.append(jax.ShapeDtypeStruct((NC, N), jnp.float32))
    scratch += [
        pltpu.VMEM((128,), jnp.float32),
        pltpu.VMEM((N,), jnp.float32),
        pltpu.VMEM_SHARED((N,), jnp.float32),
    ]

  def body(x_hbm, src_hbm, dst_hbm, agg_out, *rest):
    if with_cnt:
      (cnt_out, sidx, didx, rows, rows2, agg_sh, sem, sem_c,
       ones_v, cnt_v, cnt_sh) = rest
    else:
      (sidx, didx, rows, rows2, agg_sh, sem, sem_c) = rest
    c = lax.axis_index("c")
    s = lax.axis_index("s")
    w = c * NS + s
    zero16 = jnp.zeros((16,), jnp.float32)

    @pl.loop(0, ZC)
    def _(r):
      for j in range(D // 16):
        rows[r, pl.ds(j * 16, 16)] = zero16

    for j in range(RPT // ZC):
      pltpu.sync_copy(rows.at[pl.ds(0, ZC)],
                      agg_sh.at[pl.ds(s * RPT + j * ZC, ZC)])

    if with_cnt:
      one16 = jnp.ones((16,), jnp.float32)

      @pl.loop(0, 128 // 16)
      def _(i):
        ones_v[pl.ds(i * 16, 16)] = one16

      @pl.when(s == 0)
      def _():
        @pl.loop(0, N // 16)
        def _(i):
          cnt_v[pl.ds(i * 16, 16)] = zero16
        pltpu.sync_copy(cnt_v, cnt_sh)

    plsc.subcore_barrier()

    base = w * RPW
    bufs = (rows, rows2)

    @pl.loop(0, NBLK)
    def _(g):
      off = base + g * IJ
      pltpu.sync_copy(src_hbm.at[pl.ds(off, IJ)], sidx)
      pltpu.sync_copy(dst_hbm.at[pl.ds(off, IJ)], didx)
      gd = pltpu.async_copy(x_hbm.at[sidx.at[0]], bufs[0], sem)
      cnt_descs = []
      for j in range(IJ):
        gd.wait()
        if j + 1 < IJ:
          gd = pltpu.async_copy(x_hbm.at[sidx.at[j + 1]], bufs[(j + 1) % 2],
                                sem)
        pltpu.sync_copy(bufs[j % 2], agg_sh.at[didx.at[j]], add=True)
        if with_cnt:
          cnt_descs.append(
              pltpu.async_copy(ones_v, cnt_sh.at[didx.at[j]], sem_c,
                               add=True))
      for d in cnt_descs:
        d.wait()

    @pl.when(w < EXTRA)
    def _():
      off = RPW * NW + w
      pltpu.sync_copy(src_hbm.at[pl.ds(off, 1)], sidx.at[pl.ds(0, 1)])
      pltpu.sync_copy(dst_hbm.at[pl.ds(off, 1)], didx.at[pl.ds(0, 1)])
      pltpu.async_copy(x_hbm.at[sidx.at[0]], rows, sem).wait()
      pltpu.sync_copy(rows, agg_sh.at[didx.at[0]], add=True)
      if with_cnt:
        pltpu.sync_copy(ones_v, cnt_sh.at[didx.at[0]], add=True)

    plsc.subcore_barrier()

    for j in range(RPT // ZC):
      r0 = s * RPT + j * ZC
      pltpu.sync_copy(agg_sh.at[pl.ds(r0, ZC)], rows.at[pl.ds(0, ZC)])
      pltpu.sync_copy(rows.at[pl.ds(0, ZC)], agg_out.at[c, pl.ds(r0, ZC)])
    if with_cnt:
      @pl.when(s == 0)
      def _():
        pltpu.sync_copy(cnt_sh, cnt_v)
        pltpu.sync_copy(cnt_v, cnt_out.at[c])

  return pl.kernel(
      body, out_type=out_type, mesh=mesh, scratch_types=scratch,
      compiler_params=pltpu.CompilerParams(use_tc_tiling_on_sc=False))


_seg_sum_cnt = _seg_sum_builder(True)
_seg_sum = _seg_sum_builder(False)

_BLK = 1000


def _layer_body(aggp, cntp, x, wlT, wrT, b, o, *, relu):
  agg = aggp[0] + aggp[1]
  cnt = jnp.maximum(cntp[0] + cntp[1], 1.0)
  mean = agg / cnt
  h = (jnp.dot(mean, wlT[...], preferred_element_type=jnp.float32)
       + jnp.dot(x[...], wrT[...], preferred_element_type=jnp.float32)
       + b[...])
  if relu:
    h = jnp.maximum(h, 0.0)
  o[...] = h


def _layer(aggp, cntp, xin, wlT, wrT, b, relu):
  return pl.pallas_call(
      functools.partial(_layer_body, relu=relu),
      grid=(N // _BLK,),
      in_specs=[
          pl.BlockSpec((NC, _BLK, D), lambda i: (0, i, 0)),
          pl.BlockSpec((NC, _BLK, 1), lambda i: (0, i, 0)),
          pl.BlockSpec((_BLK, D), lambda i: (i, 0)),
          pl.BlockSpec((D, D), lambda i: (0, 0)),
          pl.BlockSpec((D, D), lambda i: (0, 0)),
          pl.BlockSpec((1, D), lambda i: (0, 0)),
      ],
      out_specs=pl.BlockSpec((_BLK, D), lambda i: (i, 0)),
      out_shape=jax.ShapeDtypeStruct((N, D), jnp.float32),
  )(aggp, cntp, xin, wlT, wrT, b)


def kernel(x, edge_index, W1l, W1r, b1, W2l, W2r, b2):
  src = edge_index[0].reshape(ROWS, 128)
  dst = edge_index[1].reshape(ROWS, 128)
  agg1, cntp = _seg_sum_cnt(x, src, dst)
  cntp = cntp.reshape(NC, N, 1)
  h = _layer(agg1, cntp, x, W1l.T, W1r.T, b1.reshape(1, D), True)
  (agg2,) = _seg_sum(h, src, dst)
  return _layer(agg2, cntp, h, W2l.T, W2r.T, b2.reshape(1, D), False)


# speedup vs baseline: 11.4026x; 1.0344x over previous
import functools

import jax
import jax.numpy as jnp
from jax import lax
from jax.experimental import pallas as pl
from jax.experimental.pallas import tpu as pltpu
from jax.experimental.pallas import tpu_sc as plsc

N = 10000
D = 128
E = 320000
NC = 2
NS = 16
NW = NC * NS
ROWS = E // 128
RPW = ROWS // NW
EXTRA = ROWS - RPW * NW
PB = 26
NPH = RPW // PB
CC = 2000
RPT = N // NS
ZC = 125


def _seg_sum_builder(with_cnt):
  mesh = plsc.VectorSubcoreMesh(core_axis_name="c", subcore_axis_name="s")
  out_type = [jax.ShapeDtypeStruct((NC, N, D), jnp.float32)]
  scratch = [
      pltpu.VMEM((PB, 128), jnp.int32),
      pltpu.VMEM((PB, 128), jnp.int32),
      pltpu.VMEM((128, D), jnp.float32),
      pltpu.VMEM((128, D), jnp.float32),
      pltpu.VMEM_SHARED((N, D), jnp.float32),
      pltpu.SemaphoreType.DMA,
      pltpu.SemaphoreType.DMA,
      pltpu.SemaphoreType.DMA,
  ]
  if with_cnt:
    out_type.append(jax.ShapeDtypeStruct((NC, N), jnp.float32))
    scratch += [
        pltpu.VMEM((128,), jnp.float32),
        pltpu.VMEM((CC,), jnp.float32),
        pltpu.VMEM_SHARED((N,), jnp.float32),
    ]

  def body(x_hbm, src_hbm, dst_hbm, agg_out, *rest):
    if with_cnt:
      (cnt_out, sidx, didx, rows, rows2, agg_sh, sem, sem_s, sem_c,
       ones_v, cnt_v, cnt_sh) = rest
    else:
      (sidx, didx, rows, rows2, agg_sh, sem, sem_s, sem_c) = rest
    c = lax.axis_index("c")
    s = lax.axis_index("s")
    w = c * NS + s
    zero16 = jnp.zeros((16,), jnp.float32)

    @pl.loop(0, ZC)
    def _(r):
      for j in range(D // 16):
        rows[r, pl.ds(j * 16, 16)] = zero16

    for j in range(RPT // ZC):
      pltpu.sync_copy(rows.at[pl.ds(0, ZC)],
                      agg_sh.at[pl.ds(s * RPT + j * ZC, ZC)])

    if with_cnt:
      one16 = jnp.ones((16,), jnp.float32)

      @pl.loop(0, 128 // 16)
      def _(i):
        ones_v[pl.ds(i * 16, 16)] = one16

      @pl.when(s == 0)
      def _():
        @pl.loop(0, CC // 16)
        def _(i):
          cnt_v[pl.ds(i * 16, 16)] = zero16
        for j in range(N // CC):
          pltpu.sync_copy(cnt_v, cnt_sh.at[pl.ds(j * CC, CC)])

    plsc.subcore_barrier()

    base = w * RPW

    def g_start(j, buf):
      pltpu.async_copy(x_hbm.at[sidx.at[j]], buf, sem)

    def g_wait(j, buf):
      pltpu.make_async_copy(x_hbm.at[sidx.at[j]], buf, sem).wait()

    def s_start(j, buf):
      pltpu.async_copy(buf, agg_sh.at[didx.at[j]], sem_s, add=True)

    def s_wait(buf):
      pltpu.make_async_copy(buf, agg_sh.at[didx.at[0]], sem_s).wait()

    for ph in range(NPH):
      off = base + ph * PB
      pltpu.sync_copy(src_hbm.at[pl.ds(off, PB)], sidx)
      pltpu.sync_copy(dst_hbm.at[pl.ds(off, PB)], didx)
      g_start(0, rows)

      @pl.loop(0, PB // 2)
      def _(t):
        j0 = 2 * t
        g_wait(j0, rows)
        @pl.when(t > 0)
        def _():
          s_wait(rows2)
        g_start(j0 + 1, rows2)
        s_start(j0, rows)
        if with_cnt:
          pltpu.async_copy(ones_v, cnt_sh.at[didx.at[j0]], sem_c, add=True)
        g_wait(j0 + 1, rows2)
        s_wait(rows)
        @pl.when(t < PB // 2 - 1)
        def _():
          g_start(j0 + 2, rows)
        s_start(j0 + 1, rows2)
        if with_cnt:
          pltpu.async_copy(ones_v, cnt_sh.at[didx.at[j0 + 1]], sem_c,
                           add=True)

      s_wait(rows2)
      if with_cnt:
        pltpu.make_async_copy(x_hbm.at[pl.ds(0, PB)],
                              rows.at[pl.ds(0, PB)], sem_c).wait()

    @pl.when(w < EXTRA)
    def _():
      off = RPW * NW + w
      pltpu.sync_copy(src_hbm.at[pl.ds(off, 1)], sidx.at[pl.ds(0, 1)])
      pltpu.sync_copy(dst_hbm.at[pl.ds(off, 1)], didx.at[pl.ds(0, 1)])
      pltpu.async_copy(x_hbm.at[sidx.at[0]], rows, sem).wait()
      pltpu.sync_copy(rows, agg_sh.at[didx.at[0]], add=True)
      if with_cnt:
        pltpu.sync_copy(ones_v, cnt_sh.at[didx.at[0]], add=True)

    plsc.subcore_barrier()

    for j in range(RPT // ZC):
      r0 = s * RPT + j * ZC
      pltpu.sync_copy(agg_sh.at[pl.ds(r0, ZC)], rows.at[pl.ds(0, ZC)])
      pltpu.sync_copy(rows.at[pl.ds(0, ZC)], agg_out.at[c, pl.ds(r0, ZC)])
    if with_cnt:
      @pl.when(s == 0)
      def _():
        for j in range(N // CC):
          pltpu.sync_copy(cnt_sh.at[pl.ds(j * CC, CC)], cnt_v)
          pltpu.sync_copy(cnt_v, cnt_out.at[c, pl.ds(j * CC, CC)])

  return pl.kernel(
      body, out_type=out_type, mesh=mesh, scratch_types=scratch,
      compiler_params=pltpu.CompilerParams(use_tc_tiling_on_sc=False))


_seg_sum_cnt = _seg_sum_builder(True)
_seg_sum = _seg_sum_builder(False)

_BLK = 1000


def _layer_body(aggp, cntp, x, wlT, wrT, b, o, *, relu):
  agg = aggp[0] + aggp[1]
  cnt = jnp.maximum(cntp[0] + cntp[1], 1.0)
  mean = agg / cnt
  h = (jnp.dot(mean, wlT[...], preferred_element_type=jnp.float32)
       + jnp.dot(x[...], wrT[...], preferred_element_type=jnp.float32)
       + b[...])
  if relu:
    h = jnp.maximum(h, 0.0)
  o[...] = h


def _layer(aggp, cntp, xin, wlT, wrT, b, relu):
  return pl.pallas_call(
      functools.partial(_layer_body, relu=relu),
      grid=(N // _BLK,),
      in_specs=[
          pl.BlockSpec((NC, _BLK, D), lambda i: (0, i, 0)),
          pl.BlockSpec((NC, _BLK, 1), lambda i: (0, i, 0)),
          pl.BlockSpec((_BLK, D), lambda i: (i, 0)),
          pl.BlockSpec((D, D), lambda i: (0, 0)),
          pl.BlockSpec((D, D), lambda i: (0, 0)),
          pl.BlockSpec((1, D), lambda i: (0, 0)),
      ],
      out_specs=pl.BlockSpec((_BLK, D), lambda i: (i, 0)),
      out_shape=jax.ShapeDtypeStruct((N, D), jnp.float32),
  )(aggp, cntp, xin, wlT, wrT, b)


def kernel(x, edge_index, W1l, W1r, b1, W2l, W2r, b2):
  src = edge_index[0].reshape(ROWS, 128)
  dst = edge_index[1].reshape(ROWS, 128)
  agg1, cntp = _seg_sum_cnt(x, src, dst)
  cntp = cntp.reshape(NC, N, 1)
  h = _layer(agg1, cntp, x, W1l.T, W1r.T, b1.reshape(1, D), True)
  (agg2,) = _seg_sum(h, src, dst)
  return _layer(agg2, cntp, h, W2l.T, W2r.T, b2.reshape(1, D), False)
